# Initial kernel scaffold; baseline (speedup 1.0000x reference)
import functools

import jax
import jax.numpy as jnp
from jax import lax
from jax.experimental import pallas as pl
from jax.experimental.pallas import tpu as pltpu
from jax.experimental.pallas import tpu_sc as plsc

LAYERS = 5
DIM = 300
DP = 304
NSL = DP // 16
NNODE = 10000
NEDGE = 160000
NG = 128
NC, NS = 2, 16
NW = NC * NS
NPT = 313
NPAD = NPT * NW
CH = 32

_f32 = jnp.float32
_i32 = jnp.int32


def _edge_body(xp, srcs, combos, dstl, starts, tab, aggr,
               tab_v, rows_v, src_v, aggr_v, comb_s, dstl_s, starts_s, sem):
    t = lax.axis_index("s") * NC + lax.axis_index("c")

    pltpu.sync_copy(starts, starts_s)
    pltpu.sync_copy(tab, tab_v)

    zero = jnp.zeros((16,), _f32)

    def zrow(i, c):
        for s in range(NSL):
            aggr_v[i, pl.ds(16 * s, 16)] = zero
        return c

    lax.fori_loop(0, NPT, zrow, 0)

    start = starts_s[t]
    end = starts_s[t + 1]
    k0 = start // CH
    k1 = (end + CH - 1) // CH

    def chunk(k, c):
        base = k * CH
        pltpu.sync_copy(srcs.at[pl.ds(base, CH)], src_v)
        pltpu.sync_copy(combos.at[pl.ds(base, CH)], comb_s)
        pltpu.sync_copy(dstl.at[pl.ds(base, CH)], dstl_s)
        pltpu.async_copy(xp.at[src_v], rows_v, sem).wait()
        lo = jnp.maximum(start, base)
        hi = jnp.minimum(end, base + CH)

        def edge(e, c2):
            r = e - base
            cb = comb_s[r]
            d = dstl_s[r]
            for s in range(NSL):
                sl = pl.ds(16 * s, 16)
                v = jnp.maximum(rows_v[r, sl] + tab_v[cb, sl], 0.0)
                plsc.addupdate(aggr_v.at[d, sl], v)
            return c2

        lax.fori_loop(lo, hi, edge, c)
        return c

    lax.fori_loop(k0, k1, chunk, 0)
    pltpu.sync_copy(aggr_v, aggr.at[pl.ds(t * NPT, NPT)])


_edge_fn = functools.partial(
    pl.kernel,
    out_type=jax.ShapeDtypeStruct((NPAD, DP), _f32),
    mesh=plsc.VectorSubcoreMesh(core_axis_name="c", subcore_axis_name="s"),
    scratch_types=[
        pltpu.VMEM((8, DP), _f32),
        pltpu.VMEM((CH, DP), _f32),
        pltpu.VMEM((CH,), _i32),
        pltpu.VMEM((NPT, DP), _f32),
        pltpu.SMEM((CH,), _i32),
        pltpu.SMEM((CH,), _i32),
        pltpu.SMEM((40,), _i32),
        pltpu.SemaphoreType.DMA,
    ],
)(_edge_body)


def _bn_relu(h, g, b):
    mu = jnp.mean(h, axis=0, keepdims=True)
    var = jnp.mean((h - mu) ** 2, axis=0, keepdims=True)
    return jnp.maximum((h - mu) * lax.rsqrt(var + 1e-5) * g + b, 0.0)


def _dense_body(xp, aggr, vn, oh,
                W1, b1, g1, be1, W2, b2, g2, be2,
                vW1, vb1, vg1, vbe1, vW2, vb2, vg2, vbe2,
                xnext, vnout):
    x = xp[...]
    u = x + aggr[...]
    h = jnp.dot(u, W1[...], preferred_element_type=_f32) + b1[...]
    h = _bn_relu(h, g1[...], be1[...])
    h = jnp.dot(h, W2[...], preferred_element_type=_f32) + b2[...]
    h = _bn_relu(h, g2[...], be2[...])
    xo = h + x
    o = oh[...]
    cnt = jnp.sum(o, axis=0, keepdims=True)
    pool = lax.dot_general(o, xo, (((0,), (0,)), ((), ())),
                           preferred_element_type=_f32)
    pool = pool / jnp.maximum(cnt, 1.0).T
    vn2 = vn[...] + pool
    v = jnp.dot(vn2, vW1[...], preferred_element_type=_f32) + vb1[...]
    v = _bn_relu(v, vg1[...], vbe1[...])
    v = jnp.dot(v, vW2[...], preferred_element_type=_f32) + vb2[...]
    v = _bn_relu(v, vg2[...], vbe2[...])
    vnout[...] = v
    xnext[...] = xo + jnp.dot(o, v, preferred_element_type=_f32)


_dense_fn = pl.pallas_call(
    _dense_body,
    out_shape=[
        jax.ShapeDtypeStruct((NNODE, DP), _f32),
        jax.ShapeDtypeStruct((NG, DP), _f32),
    ],
)


def _final_body(xp, aggr, oh, W1, b1, g1, be1, W2, b2, g2, be2, lW, lb, out):
    x = xp[...]
    u = x + aggr[...]
    h = jnp.dot(u, W1[...], preferred_element_type=_f32) + b1[...]
    h = _bn_relu(h, g1[...], be1[...])
    h = jnp.dot(h, W2[...], preferred_element_type=_f32) + b2[...]
    h = _bn_relu(h, g2[...], be2[...])
    xo = h + x
    o = oh[...]
    cnt = jnp.sum(o, axis=0, keepdims=True)
    pool = lax.dot_general(o, xo, (((0,), (0,)), ((), ())),
                           preferred_element_type=_f32)
    pool = pool / jnp.maximum(cnt, 1.0).T
    out[...] = jnp.dot(pool, lW[...], preferred_element_type=_f32) + lb[...]


_final_fn = pl.pallas_call(
    _final_body,
    out_shape=jax.ShapeDtypeStruct((NG, 8), _f32),
)


def _pad_mat(w):
    return jnp.pad(w, ((0, DP - DIM), (0, DP - DIM)))


def _pad_vec(v):
    return jnp.pad(v, (0, DP - DIM))[None, :]


def kernel(x, edge_index, edge_attr, batch, params):
    p = params
    src = edge_index[0].astype(_i32)
    dst = edge_index[1].astype(_i32)
    combo = (edge_attr[:, 0] * 4 + edge_attr[:, 1] * 2
             + edge_attr[:, 2]).astype(_i32)
    perm = jnp.argsort(dst)
    srcs = src[perm]
    combos = combo[perm]
    dsts = dst[perm]
    dstl = (dsts % NPT).astype(_i32)
    bounds = (jnp.arange(1, NW + 1) * NPT).astype(_i32)
    starts = jnp.concatenate(
        [jnp.zeros((1,), _i32), jnp.searchsorted(dsts, bounds).astype(_i32)])
    starts = jnp.pad(starts, (0, 7))

    tab = (p['bond_t0'][:2, None, None, :] + p['bond_t1'][None, :2, None, :]
           + p['bond_t2'][None, None, :2, :]).reshape(8, DIM)
    tab = jnp.pad(tab, ((0, 0), (0, DP - DIM)))

    oh = (batch[:, None] == jnp.arange(NG, dtype=batch.dtype)[None, :]).astype(_f32)

    row0 = jnp.pad((p['const_x'] + p['vn_emb'])[0], (0, DP - DIM))
    xp = jnp.broadcast_to(row0, (NNODE, DP))
    vn = jnp.broadcast_to(jnp.pad(p['vn_emb'][0], (0, DP - DIM)), (NG, DP))

    for l in range(LAYERS):
        aggr = _edge_fn(xp, srcs, combos, dstl, starts, tab)[:NNODE]
        cw1 = _pad_mat(p['conv_W1'][l])
        cb1 = _pad_vec(p['conv_b1'][l])
        cg1 = _pad_vec(p['conv_g1'][l])
        cbe1 = _pad_vec(p['conv_be1'][l])
        cw2 = _pad_mat(p['conv_W2'][l])
        cb2 = _pad_vec(p['conv_b2'][l])
        cg2 = _pad_vec(p['bn_g'][l])
        cbe2 = _pad_vec(p['bn_b'][l])
        if l < LAYERS - 1:
            xp, vn = _dense_fn(
                xp, aggr, vn, oh,
                cw1, cb1, cg1, cbe1, cw2, cb2, cg2, cbe2,
                _pad_mat(p['vn_W1'][l]), _pad_vec(p['vn_b1'][l]),
                _pad_vec(p['vn_g1'][l]), _pad_vec(p['vn_be1'][l]),
                _pad_mat(p['vn_W2'][l]), _pad_vec(p['vn_b2'][l]),
                _pad_vec(p['vn_g2'][l]), _pad_vec(p['vn_be2'][l]))
        else:
            lw = jnp.pad(p['lin_W'], ((0, DP - DIM), (0, 7)))
            lb = jnp.pad(p['lin_b'], (0, 7))[None, :]
            out = _final_fn(xp, aggr, oh,
                            cw1, cb1, cg1, cbe1, cw2, cb2, cg2, cbe2, lw, lb)
    return out[:, :1]


# speedup vs baseline: 1.6644x; 1.6644x over previous
import functools

import jax
import jax.numpy as jnp
from jax import lax
from jax.experimental import pallas as pl
from jax.experimental.pallas import tpu as pltpu
from jax.experimental.pallas import tpu_sc as plsc

LAYERS = 5
DIM = 300
DP = 304
NSL = DP // 16
NNODE = 10000
NEDGE = 160000
NG = 128
NC, NS = 2, 16
NW = NC * NS
NPT = 320
NPAD = NPT * NW
CH = 32

BN_ = 1000
NB = NNODE // BN_

_f32 = jnp.float32
_i32 = jnp.int32


def _edge_body(xp, srcs, combos, dstl, starts, tab, aggr,
               tab_v, rows_v, src_v, aggr_v, comb_s, dstl_s, starts_s, sem):
    t = lax.axis_index("s") * NC + lax.axis_index("c")

    pltpu.sync_copy(starts, starts_s)
    pltpu.sync_copy(tab, tab_v)

    def _scal(ref, i):
        return ref[pl.ds(i, 16)][0]

    zero = jnp.zeros((16,), _f32)

    def zrow(i, c):
        for s in range(NSL):
            aggr_v[i, pl.ds(16 * s, 16)] = zero
        return c

    lax.fori_loop(0, NPT, zrow, 0)

    start = _scal(starts_s, t)
    end = _scal(starts_s, t + 1)
    k0 = start // CH
    k1 = (end + CH - 1) // CH

    def chunk(k, c):
        base = k * CH
        pltpu.sync_copy(srcs.at[pl.ds(base, CH)], src_v)
        pltpu.sync_copy(combos.at[pl.ds(base, CH)], comb_s.at[pl.ds(0, CH)])
        pltpu.sync_copy(dstl.at[pl.ds(base, CH)], dstl_s.at[pl.ds(0, CH)])
        pltpu.async_copy(xp.at[src_v], rows_v, sem).wait()
        lo = jnp.maximum(start, base)
        hi = jnp.minimum(end, base + CH)

        def edge(e, c2):
            r = e - base
            cb = _scal(comb_s, r)
            d = _scal(dstl_s, r)
            for s in range(NSL):
                sl = pl.ds(16 * s, 16)
                v = jnp.maximum(rows_v[r, sl] + tab_v[cb, sl], 0.0)
                plsc.addupdate(aggr_v.at[d, sl], v)
            return c2

        lax.fori_loop(lo, hi, edge, c)
        return c

    lax.fori_loop(k0, k1, chunk, 0)
    pltpu.sync_copy(aggr_v, aggr.at[pl.ds(t * NPT, NPT)])


@functools.cache
def _edge_fn():
    return functools.partial(
        pl.kernel,
        out_type=jax.ShapeDtypeStruct((NPAD, DP), _f32),
        mesh=plsc.VectorSubcoreMesh(core_axis_name="c", subcore_axis_name="s",
                                    num_cores=NC, num_subcores=NS),
        compiler_params=pltpu.CompilerParams(use_tc_tiling_on_sc=False),
        scratch_types=[
            pltpu.VMEM((8, DP), _f32),
            pltpu.VMEM((CH, DP), _f32),
            pltpu.VMEM((CH,), _i32),
            pltpu.VMEM((NPT, DP), _f32),
            pltpu.VMEM((CH + 16,), _i32),
            pltpu.VMEM((CH + 16,), _i32),
            pltpu.VMEM((48,), _i32),
            pltpu.SemaphoreType.DMA,
        ],
    )(_edge_body)


_ARB = pltpu.CompilerParams(dimension_semantics=("arbitrary",))


def _blk(r, c):
    return pl.BlockSpec((r, c), lambda i: (i, 0))


def _rep(r, c):
    return pl.BlockSpec((r, c), lambda i: (0, 0))


def _acc_stats(acc, h, i):
    bmu = jnp.mean(h, axis=0, keepdims=True)
    bM2 = jnp.sum((h - bmu) ** 2, axis=0, keepdims=True)

    @pl.when(i == 0)
    def _():
        acc[0:1, :] = bmu
        acc[1:2, :] = bM2

    @pl.when(i > 0)
    def _():
        na = i.astype(_f32) * BN_
        tot = na + BN_
        delta = bmu - acc[0:1, :]
        acc[0:1, :] += delta * (BN_ / tot)
        acc[1:2, :] += bM2 + delta * delta * (na * (BN_ / tot))


def _k1_body(xp, aggr, W1, b1, h1, stats, acc):
    i = pl.program_id(0)
    u = xp[...] + aggr[...]
    h = jnp.dot(u, W1[...], preferred_element_type=_f32) + b1[...]
    h1[...] = h
    _acc_stats(acc, h, i)

    @pl.when(i == NB - 1)
    def _():
        stats[...] = acc[...]


_k1 = pl.pallas_call(
    _k1_body,
    grid=(NB,),
    in_specs=[_blk(BN_, DP), _blk(BN_, DP), _rep(DP, DP), _rep(1, DP)],
    out_specs=[_blk(BN_, DP), _rep(8, DP)],
    out_shape=[jax.ShapeDtypeStruct((NNODE, DP), _f32),
               jax.ShapeDtypeStruct((8, DP), _f32)],
    scratch_shapes=[pltpu.VMEM((8, DP), _f32)],
    compiler_params=_ARB,
)


def _norm(stats, g, b):
    mu = stats[0:1, :]
    var = stats[1:2, :] * (1.0 / NNODE)
    scale = lax.rsqrt(var + 1e-5) * g
    return mu, scale, b


def _k2_body(h1, stats1, g1, be1, W2, b2, h2, stats, acc):
    i = pl.program_id(0)
    mu, scale, b = _norm(stats1[...], g1[...], be1[...])
    hn = jnp.maximum((h1[...] - mu) * scale + b, 0.0)
    h = jnp.dot(hn, W2[...], preferred_element_type=_f32) + b2[...]
    h2[...] = h
    _acc_stats(acc, h, i)

    @pl.when(i == NB - 1)
    def _():
        stats[...] = acc[...]


_k2 = pl.pallas_call(
    _k2_body,
    grid=(NB,),
    in_specs=[_blk(BN_, DP), _rep(8, DP), _rep(1, DP), _rep(1, DP),
              _rep(DP, DP), _rep(1, DP)],
    out_specs=[_blk(BN_, DP), _rep(8, DP)],
    out_shape=[jax.ShapeDtypeStruct((NNODE, DP), _f32),
               jax.ShapeDtypeStruct((8, DP), _f32)],
    scratch_shapes=[pltpu.VMEM((8, DP), _f32)],
    compiler_params=_ARB,
)


def _k3_body(h2, stats2, g2, be2, xp, oh, xo, pool, cnt, accp, accc):
    i = pl.program_id(0)
    mu, scale, b = _norm(stats2[...], g2[...], be2[...])
    xov = jnp.maximum((h2[...] - mu) * scale + b, 0.0) + xp[...]
    xo[...] = xov
    o = oh[...]

    @pl.when(i == 0)
    def _():
        accp[...] = jnp.zeros_like(accp)
        accc[...] = jnp.zeros_like(accc)

    accp[...] += lax.dot_general(o, xov, (((0,), (0,)), ((), ())),
                                 preferred_element_type=_f32,
                                 precision=lax.Precision.HIGHEST)
    accc[...] += lax.dot_general(o, jnp.ones((BN_, 8), _f32),
                                 (((0,), (0,)), ((), ())),
                                 preferred_element_type=_f32,
                                 precision=lax.Precision.HIGHEST)

    @pl.when(i == NB - 1)
    def _():
        pool[...] = accp[...]
        cnt[...] = accc[...]


_k3 = pl.pallas_call(
    _k3_body,
    grid=(NB,),
    in_specs=[_blk(BN_, DP), _rep(8, DP), _rep(1, DP), _rep(1, DP),
              _blk(BN_, DP), _blk(BN_, NG)],
    out_specs=[_blk(BN_, DP), _rep(NG, DP), _rep(NG, 8)],
    out_shape=[jax.ShapeDtypeStruct((NNODE, DP), _f32),
               jax.ShapeDtypeStruct((NG, DP), _f32),
               jax.ShapeDtypeStruct((NG, 8), _f32)],
    scratch_shapes=[pltpu.VMEM((NG, DP), _f32), pltpu.VMEM((NG, 8), _f32)],
    compiler_params=_ARB,
)


def _bn_relu_rows(h, g, b, n):
    mu = jnp.mean(h, axis=0, keepdims=True)
    var = jnp.mean((h - mu) ** 2, axis=0, keepdims=True)
    return jnp.maximum((h - mu) * lax.rsqrt(var + 1e-5) * g + b, 0.0)


def _k4_body(pool, cnt, vn, vW1, vb1, vg1, vbe1, vW2, vb2, vg2, vbe2, vnout):
    invc = 1.0 / jnp.maximum(cnt[...][:, 0:1], 1.0)
    vn2 = vn[...] + pool[...] * invc
    v = jnp.dot(vn2, vW1[...], preferred_element_type=_f32) + vb1[...]
    v = _bn_relu_rows(v, vg1[...], vbe1[...], NG)
    v = jnp.dot(v, vW2[...], preferred_element_type=_f32) + vb2[...]
    v = _bn_relu_rows(v, vg2[...], vbe2[...], NG)
    vnout[...] = v


_k4 = pl.pallas_call(
    _k4_body,
    out_shape=jax.ShapeDtypeStruct((NG, DP), _f32),
)


def _k5_body(xo, oh, vn3, xnext):
    xnext[...] = xo[...] + jnp.dot(oh[...], vn3[...],
                                   preferred_element_type=_f32,
                                   precision=lax.Precision.HIGHEST)


_k5 = pl.pallas_call(
    _k5_body,
    grid=(NB,),
    in_specs=[_blk(BN_, DP), _blk(BN_, NG), _rep(NG, DP)],
    out_specs=_blk(BN_, DP),
    out_shape=jax.ShapeDtypeStruct((NNODE, DP), _f32),
    compiler_params=_ARB,
)


def _k4f_body(pool, cnt, lW, lb, out):
    invc = 1.0 / jnp.maximum(cnt[...][:, 0:1], 1.0)
    out[...] = jnp.dot(pool[...] * invc, lW[...],
                       preferred_element_type=_f32) + lb[...]


_k4f = pl.pallas_call(
    _k4f_body,
    out_shape=jax.ShapeDtypeStruct((NG, 8), _f32),
)


def _pad_mat(w):
    return jnp.pad(w, ((0, DP - DIM), (0, DP - DIM)))


def _pad_vec(v):
    return jnp.pad(v, (0, DP - DIM))[None, :]


def kernel(x, edge_index, edge_attr, batch, params):
    p = params
    src = edge_index[0].astype(_i32)
    dst = edge_index[1].astype(_i32)
    combo = (edge_attr[:, 0] * 4 + edge_attr[:, 1] * 2
             + edge_attr[:, 2]).astype(_i32)
    perm = jnp.argsort(dst)
    srcs = src[perm]
    combos = combo[perm]
    dsts = dst[perm]
    dstl = (dsts % NPT).astype(_i32)
    bounds = (jnp.arange(1, NW + 1) * NPT).astype(_i32)
    starts = jnp.concatenate(
        [jnp.zeros((1,), _i32), jnp.searchsorted(dsts, bounds).astype(_i32)])
    starts = jnp.pad(starts, (0, 15))

    tab = (p['bond_t0'][:2, None, None, :] + p['bond_t1'][None, :2, None, :]
           + p['bond_t2'][None, None, :2, :]).reshape(8, DIM)
    tab = jnp.pad(tab, ((0, 0), (0, DP - DIM)))

    oh = (batch[:, None] == jnp.arange(NG, dtype=batch.dtype)[None, :]).astype(_f32)

    row0 = jnp.pad((p['const_x'] + p['vn_emb'])[0], (0, DP - DIM))
    xp = jnp.broadcast_to(row0, (NNODE, DP))

    vn = jnp.broadcast_to(jnp.pad(p['vn_emb'][0], (0, DP - DIM)), (NG, DP))

    for l in range(LAYERS):
        aggr = _edge_fn()(xp, srcs, combos, dstl, starts, tab)
        h1, st1 = _k1(xp, aggr[:NNODE], _pad_mat(p['conv_W1'][l]),
                      _pad_vec(p['conv_b1'][l]))
        h2, st2 = _k2(h1, st1, _pad_vec(p['conv_g1'][l]),
                      _pad_vec(p['conv_be1'][l]), _pad_mat(p['conv_W2'][l]),
                      _pad_vec(p['conv_b2'][l]))
        xo, pool, cnt = _k3(h2, st2, _pad_vec(p['bn_g'][l]),
                            _pad_vec(p['bn_b'][l]), xp, oh)
        if l < LAYERS - 1:
            vn = _k4(pool, cnt, vn,
                     _pad_mat(p['vn_W1'][l]), _pad_vec(p['vn_b1'][l]),
                     _pad_vec(p['vn_g1'][l]), _pad_vec(p['vn_be1'][l]),
                     _pad_mat(p['vn_W2'][l]), _pad_vec(p['vn_b2'][l]),
                     _pad_vec(p['vn_g2'][l]), _pad_vec(p['vn_be2'][l]))
            xp = _k5(xo, oh, vn)
        else:
            lw = jnp.pad(p['lin_W'], ((0, DP - DIM), (0, 7)))
            lb = jnp.pad(p['lin_b'], (0, 7))[None, :]
            out = _k4f(pool, cnt, lw, lb)
    return out[:, :1]


# speedup vs baseline: 2.3382x; 1.4048x over previous
import functools

import jax
import jax.numpy as jnp
from jax import lax
from jax.experimental import pallas as pl
from jax.experimental.pallas import tpu as pltpu
from jax.experimental.pallas import tpu_sc as plsc

LAYERS = 5
DIM = 300
DP = 304
NSL = DP // 16
NNODE = 10000
NEDGE = 160000
NG = 128
NC, NS = 2, 16
NW = NC * NS
NPT = 320
NPAD = NPT * NW
CH = 32

BN_ = 1000
NB = NNODE // BN_

_f32 = jnp.float32
_i32 = jnp.int32


def _edge_body(xp, ew, starts, tab, aggr,
               tab_v, rows_v, src_v, ew_v, aggr_v, starts_s, sem_i, sem_r):
    t = lax.axis_index("s") * NC + lax.axis_index("c")

    pltpu.sync_copy(starts, starts_s)
    pltpu.sync_copy(tab, tab_v)

    def _scal(ref, i):
        return ref[pl.ds(i, 16)][0]

    def _scal2(ref, j, i):
        return ref[j, pl.ds(i, 16)][0]

    zero = jnp.zeros((16,), _f32)

    def zrow(i, c):
        for s in range(NSL):
            aggr_v[i, pl.ds(16 * s, 16)] = zero
        return c

    lax.fori_loop(0, NPT, zrow, 0)

    start = _scal(starts_s, t)
    end = _scal(starts_s, t + 1)
    k0 = start // CH
    k1 = (end + CH - 1) // CH
    nch = k1 - k0

    def _idx_copy(k):
        return pltpu.make_async_copy(
            ew.at[pl.ds(k * CH, CH)],
            ew_v.at[k & 3, pl.ds(0, CH)],
            sem_i.at[k & 3])

    def _issue_gather(k):
        sl = k & 1
        for j in range(CH // 16):
            w = ew_v[k & 3, pl.ds(16 * j, 16)]
            src_v[sl, pl.ds(16 * j, 16)] = w >> 12
        pltpu.async_copy(xp.at[src_v.at[sl]], rows_v.at[sl], sem_r.at[sl])

    @pl.when(nch > 0)
    def _():
        _idx_copy(k0).start()

    @pl.when(nch > 1)
    def _():
        _idx_copy(k0 + 1).start()

    @pl.when(nch > 0)
    def _():
        _idx_copy(k0).wait()
        _issue_gather(k0)

    def chunk(k, c):
        base = k * CH
        sl = k & 1
        sle = k & 3

        @pl.when(k + 2 < k1)
        def _():
            _idx_copy(k + 2).start()

        @pl.when(k + 1 < k1)
        def _():
            _idx_copy(k + 1).wait()
            _issue_gather(k + 1)

        pltpu.make_async_copy(xp.at[src_v.at[sl]], rows_v.at[sl],
                              sem_r.at[sl]).wait()

        lo = jnp.maximum(start, base)
        hi = jnp.minimum(end, base + CH)

        def edge(e, c2):
            r = e - base
            w = _scal2(ew_v, sle, r)
            cb = (w >> 9) & 7
            d = w & 511
            for s in range(NSL):
                slc = pl.ds(16 * s, 16)
                v = jnp.maximum(rows_v[sl, r, slc] + tab_v[cb, slc], 0.0)
                plsc.addupdate(aggr_v.at[d, slc], v)
            return c2

        lax.fori_loop(lo, hi, edge, c)
        return c

    lax.fori_loop(k0, k1, chunk, 0)
    pltpu.sync_copy(aggr_v, aggr.at[pl.ds(t * NPT, NPT)])


@functools.cache
def _edge_fn():
    return functools.partial(
        pl.kernel,
        out_type=jax.ShapeDtypeStruct((NPAD, DP), _f32),
        mesh=plsc.VectorSubcoreMesh(core_axis_name="c", subcore_axis_name="s",
                                    num_cores=NC, num_subcores=NS),
        compiler_params=pltpu.CompilerParams(use_tc_tiling_on_sc=False),
        scratch_types=[
            pltpu.VMEM((8, DP), _f32),
            pltpu.VMEM((2, CH, DP), _f32),
            pltpu.VMEM((2, CH), _i32),
            pltpu.VMEM((4, CH + 16), _i32),
            pltpu.VMEM((NPT, DP), _f32),
            pltpu.VMEM((48,), _i32),
            pltpu.SemaphoreType.DMA((4,)),
            pltpu.SemaphoreType.DMA((2,)),
        ],
    )(_edge_body)


_ARB = pltpu.CompilerParams(dimension_semantics=("arbitrary",))


def _blk(r, c):
    return pl.BlockSpec((r, c), lambda i: (i, 0))


def _rep(r, c):
    return pl.BlockSpec((r, c), lambda i: (0, 0))


def _acc_stats(acc, h, i):
    bmu = jnp.mean(h, axis=0, keepdims=True)
    bM2 = jnp.sum((h - bmu) ** 2, axis=0, keepdims=True)

    @pl.when(i == 0)
    def _():
        acc[0:1, :] = bmu
        acc[1:2, :] = bM2

    @pl.when(i > 0)
    def _():
        na = i.astype(_f32) * BN_
        tot = na + BN_
        delta = bmu - acc[0:1, :]
        acc[0:1, :] += delta * (BN_ / tot)
        acc[1:2, :] += bM2 + delta * delta * (na * (BN_ / tot))


def _k1_body(xp, aggr, W1, b1, h1, stats, acc):
    i = pl.program_id(0)
    u = xp[...] + aggr[...]
    h = jnp.dot(u, W1[...], preferred_element_type=_f32) + b1[...]
    h1[...] = h
    _acc_stats(acc, h, i)

    @pl.when(i == NB - 1)
    def _():
        stats[...] = acc[...]


_k1 = pl.pallas_call(
    _k1_body,
    grid=(NB,),
    in_specs=[_blk(BN_, DP), _blk(BN_, DP), _rep(DP, DP), _rep(1, DP)],
    out_specs=[_blk(BN_, DP), _rep(8, DP)],
    out_shape=[jax.ShapeDtypeStruct((NNODE, DP), _f32),
               jax.ShapeDtypeStruct((8, DP), _f32)],
    scratch_shapes=[pltpu.VMEM((8, DP), _f32)],
    compiler_params=_ARB,
)


def _norm(stats, g, b):
    mu = stats[0:1, :]
    var = stats[1:2, :] * (1.0 / NNODE)
    scale = lax.rsqrt(var + 1e-5) * g
    return mu, scale, b


def _k2_body(h1, stats1, g1, be1, W2, b2, h2, stats, acc):
    i = pl.program_id(0)
    mu, scale, b = _norm(stats1[...], g1[...], be1[...])
    hn = jnp.maximum((h1[...] - mu) * scale + b, 0.0)
    h = jnp.dot(hn, W2[...], preferred_element_type=_f32) + b2[...]
    h2[...] = h
    _acc_stats(acc, h, i)

    @pl.when(i == NB - 1)
    def _():
        stats[...] = acc[...]


_k2 = pl.pallas_call(
    _k2_body,
    grid=(NB,),
    in_specs=[_blk(BN_, DP), _rep(8, DP), _rep(1, DP), _rep(1, DP),
              _rep(DP, DP), _rep(1, DP)],
    out_specs=[_blk(BN_, DP), _rep(8, DP)],
    out_shape=[jax.ShapeDtypeStruct((NNODE, DP), _f32),
               jax.ShapeDtypeStruct((8, DP), _f32)],
    scratch_shapes=[pltpu.VMEM((8, DP), _f32)],
    compiler_params=_ARB,
)


def _k3_body(h2, stats2, g2, be2, xp, oh, xo, pool, cnt, accp, accc):
    i = pl.program_id(0)
    mu, scale, b = _norm(stats2[...], g2[...], be2[...])
    xov = jnp.maximum((h2[...] - mu) * scale + b, 0.0) + xp[...]
    xo[...] = xov
    o = oh[...]

    @pl.when(i == 0)
    def _():
        accp[...] = jnp.zeros_like(accp)
        accc[...] = jnp.zeros_like(accc)

    accp[...] += lax.dot_general(o, xov, (((0,), (0,)), ((), ())),
                                 preferred_element_type=_f32,
                                 precision=lax.Precision.HIGHEST)
    accc[...] += lax.dot_general(o, jnp.ones((BN_, 8), _f32),
                                 (((0,), (0,)), ((), ())),
                                 preferred_element_type=_f32,
                                 precision=lax.Precision.HIGHEST)

    @pl.when(i == NB - 1)
    def _():
        pool[...] = accp[...]
        cnt[...] = accc[...]


_k3 = pl.pallas_call(
    _k3_body,
    grid=(NB,),
    in_specs=[_blk(BN_, DP), _rep(8, DP), _rep(1, DP), _rep(1, DP),
              _blk(BN_, DP), _blk(BN_, NG)],
    out_specs=[_blk(BN_, DP), _rep(NG, DP), _rep(NG, 8)],
    out_shape=[jax.ShapeDtypeStruct((NNODE, DP), _f32),
               jax.ShapeDtypeStruct((NG, DP), _f32),
               jax.ShapeDtypeStruct((NG, 8), _f32)],
    scratch_shapes=[pltpu.VMEM((NG, DP), _f32), pltpu.VMEM((NG, 8), _f32)],
    compiler_params=_ARB,
)


def _bn_relu_rows(h, g, b, n):
    mu = jnp.mean(h, axis=0, keepdims=True)
    var = jnp.mean((h - mu) ** 2, axis=0, keepdims=True)
    return jnp.maximum((h - mu) * lax.rsqrt(var + 1e-5) * g + b, 0.0)


def _k4_body(pool, cnt, vn, vW1, vb1, vg1, vbe1, vW2, vb2, vg2, vbe2, vnout):
    invc = 1.0 / jnp.maximum(cnt[...][:, 0:1], 1.0)
    vn2 = vn[...] + pool[...] * invc
    v = jnp.dot(vn2, vW1[...], preferred_element_type=_f32) + vb1[...]
    v = _bn_relu_rows(v, vg1[...], vbe1[...], NG)
    v = jnp.dot(v, vW2[...], preferred_element_type=_f32) + vb2[...]
    v = _bn_relu_rows(v, vg2[...], vbe2[...], NG)
    vnout[...] = v


_k4 = pl.pallas_call(
    _k4_body,
    out_shape=jax.ShapeDtypeStruct((NG, DP), _f32),
)


def _k5_body(xo, oh, vn3, xnext):
    xnext[...] = xo[...] + jnp.dot(oh[...], vn3[...],
                                   preferred_element_type=_f32,
                                   precision=lax.Precision.HIGHEST)


_k5 = pl.pallas_call(
    _k5_body,
    grid=(NB,),
    in_specs=[_blk(BN_, DP), _blk(BN_, NG), _rep(NG, DP)],
    out_specs=_blk(BN_, DP),
    out_shape=jax.ShapeDtypeStruct((NNODE, DP), _f32),
    compiler_params=_ARB,
)


def _k4f_body(pool, cnt, lW, lb, out):
    invc = 1.0 / jnp.maximum(cnt[...][:, 0:1], 1.0)
    out[...] = jnp.dot(pool[...] * invc, lW[...],
                       preferred_element_type=_f32) + lb[...]


_k4f = pl.pallas_call(
    _k4f_body,
    out_shape=jax.ShapeDtypeStruct((NG, 8), _f32),
)


def _pad_mat(w):
    return jnp.pad(w, ((0, DP - DIM), (0, DP - DIM)))


def _pad_vec(v):
    return jnp.pad(v, (0, DP - DIM))[None, :]


def kernel(x, edge_index, edge_attr, batch, params):
    p = params
    src = edge_index[0].astype(_i32)
    dst = edge_index[1].astype(_i32)
    combo = (edge_attr[:, 0] * 4 + edge_attr[:, 1] * 2
             + edge_attr[:, 2]).astype(_i32)
    perm = jnp.argsort(dst)
    srcs = src[perm]
    combos = combo[perm]
    dsts = dst[perm]
    dstl = (dsts % NPT).astype(_i32)
    ew = srcs * 4096 + combos * 512 + dstl
    bounds = (jnp.arange(1, NW + 1) * NPT).astype(_i32)
    starts = jnp.concatenate(
        [jnp.zeros((1,), _i32), jnp.searchsorted(dsts, bounds).astype(_i32)])
    starts = jnp.pad(starts, (0, 15))

    tab = (p['bond_t0'][:2, None, None, :] + p['bond_t1'][None, :2, None, :]
           + p['bond_t2'][None, None, :2, :]).reshape(8, DIM)
    tab = jnp.pad(tab, ((0, 0), (0, DP - DIM)))

    oh = (batch[:, None] == jnp.arange(NG, dtype=batch.dtype)[None, :]).astype(_f32)

    row0 = jnp.pad((p['const_x'] + p['vn_emb'])[0], (0, DP - DIM))
    xp = jnp.broadcast_to(row0, (NNODE, DP))

    vn = jnp.broadcast_to(jnp.pad(p['vn_emb'][0], (0, DP - DIM)), (NG, DP))

    for l in range(LAYERS):
        aggr = _edge_fn()(xp, ew, starts, tab)
        h1, st1 = _k1(xp, aggr[:NNODE], _pad_mat(p['conv_W1'][l]),
                      _pad_vec(p['conv_b1'][l]))
        h2, st2 = _k2(h1, st1, _pad_vec(p['conv_g1'][l]),
                      _pad_vec(p['conv_be1'][l]), _pad_mat(p['conv_W2'][l]),
                      _pad_vec(p['conv_b2'][l]))
        xo, pool, cnt = _k3(h2, st2, _pad_vec(p['bn_g'][l]),
                            _pad_vec(p['bn_b'][l]), xp, oh)
        if l < LAYERS - 1:
            vn = _k4(pool, cnt, vn,
                     _pad_mat(p['vn_W1'][l]), _pad_vec(p['vn_b1'][l]),
                     _pad_vec(p['vn_g1'][l]), _pad_vec(p['vn_be1'][l]),
                     _pad_mat(p['vn_W2'][l]), _pad_vec(p['vn_b2'][l]),
                     _pad_vec(p['vn_g2'][l]), _pad_vec(p['vn_be2'][l]))
            xp = _k5(xo, oh, vn)
        else:
            lw = jnp.pad(p['lin_W'], ((0, DP - DIM), (0, 7)))
            lb = jnp.pad(p['lin_b'], (0, 7))[None, :]
            out = _k4f(pool, cnt, lw, lb)
    return out[:, :1]


# speedup vs baseline: 5.0187x; 2.1464x over previous
import functools

import jax
import jax.numpy as jnp
from jax import lax
from jax.experimental import pallas as pl
from jax.experimental.pallas import tpu as pltpu
from jax.experimental.pallas import tpu_sc as plsc

LAYERS = 5
DIM = 300
DP = 304
NSL = DP // 16
NNODE = 10000
NEDGE = 160000
NG = 128
NC, NS = 2, 16
NW = NC * NS
NPT = 320
NPAD = NPT * NW
CH = 32

BN_ = 1000
NB = NNODE // BN_

_f32 = jnp.float32
_i32 = jnp.int32


def _edge_body(xp, ew, starts, tab, aggr,
               tab_v, rows_v, src_v, ew_v, aggr_v, starts_s, sem_i, sem_r):
    t = lax.axis_index("s") * NC + lax.axis_index("c")

    pltpu.sync_copy(starts, starts_s)
    pltpu.sync_copy(tab, tab_v)

    def _scal(ref, i):
        return ref[pl.ds(i, 16)][0]

    def _scal2(ref, j, i):
        return ref[j, pl.ds(i, 16)][0]

    zero = jnp.zeros((16,), _f32)

    def zrow(i, c):
        for s in range(NSL):
            aggr_v[i, pl.ds(16 * s, 16)] = zero
        return c

    lax.fori_loop(0, NPT, zrow, 0)

    start = _scal(starts_s, t)
    end = _scal(starts_s, t + 1)
    k0 = start // CH
    k1 = (end + CH - 1) // CH
    nch = k1 - k0

    def _idx_copy(k):
        return pltpu.make_async_copy(
            ew.at[pl.ds(k * CH, CH)],
            ew_v.at[k & 3, pl.ds(0, CH)],
            sem_i.at[k & 3])

    def _issue_gather(k):
        sl = k & 1
        for j in range(CH // 16):
            w = ew_v[k & 3, pl.ds(16 * j, 16)]
            src_v[sl, pl.ds(16 * j, 16)] = w >> 12
        pltpu.async_copy(xp.at[src_v.at[sl]], rows_v.at[sl], sem_r.at[sl])

    @pl.when(nch > 0)
    def _():
        _idx_copy(k0).start()

    @pl.when(nch > 1)
    def _():
        _idx_copy(k0 + 1).start()

    @pl.when(nch > 0)
    def _():
        _idx_copy(k0).wait()
        _issue_gather(k0)

    def chunk(k, c):
        base = k * CH
        sl = k & 1
        sle = k & 3

        @pl.when(k + 2 < k1)
        def _():
            _idx_copy(k + 2).start()

        @pl.when(k + 1 < k1)
        def _():
            _idx_copy(k + 1).wait()
            _issue_gather(k + 1)

        pltpu.make_async_copy(xp.at[src_v.at[sl]], rows_v.at[sl],
                              sem_r.at[sl]).wait()

        lo = jnp.maximum(start, base)
        hi = jnp.minimum(end, base + CH)

        @plsc.parallel_loop(lo, hi, unroll=2)
        def edge(e):
            r = e - base
            w = _scal2(ew_v, sle, r)
            cb = (w >> 9) & 7
            d = w & 511
            for s in range(NSL):
                slc = pl.ds(16 * s, 16)
                v = jnp.maximum(rows_v[sl, r, slc] + tab_v[cb, slc], 0.0)
                plsc.addupdate(aggr_v.at[d, slc], v)

        return c

    lax.fori_loop(k0, k1, chunk, 0)
    pltpu.sync_copy(aggr_v, aggr.at[pl.ds(t * NPT, NPT)])


@functools.cache
def _edge_fn():
    return functools.partial(
        pl.kernel,
        out_type=jax.ShapeDtypeStruct((NPAD, DP), _f32),
        mesh=plsc.VectorSubcoreMesh(core_axis_name="c", subcore_axis_name="s",
                                    num_cores=NC, num_subcores=NS),
        compiler_params=pltpu.CompilerParams(use_tc_tiling_on_sc=False),
        scratch_types=[
            pltpu.VMEM((8, DP), _f32),
            pltpu.VMEM((2, CH, DP), _f32),
            pltpu.VMEM((2, CH), _i32),
            pltpu.VMEM((4, CH + 16), _i32),
            pltpu.VMEM((NPT, DP), _f32),
            pltpu.VMEM((48,), _i32),
            pltpu.SemaphoreType.DMA((4,)),
            pltpu.SemaphoreType.DMA((2,)),
        ],
    )(_edge_body)


_ARB = pltpu.CompilerParams(dimension_semantics=("arbitrary",))


def _blk(r, c):
    return pl.BlockSpec((r, c), lambda i: (i, 0))


def _rep(r, c):
    return pl.BlockSpec((r, c), lambda i: (0, 0))


def _acc_stats(acc, h, i):
    bmu = jnp.mean(h, axis=0, keepdims=True)
    bM2 = jnp.sum((h - bmu) ** 2, axis=0, keepdims=True)

    @pl.when(i == 0)
    def _():
        acc[0:1, :] = bmu
        acc[1:2, :] = bM2

    @pl.when(i > 0)
    def _():
        na = i.astype(_f32) * BN_
        tot = na + BN_
        delta = bmu - acc[0:1, :]
        acc[0:1, :] += delta * (BN_ / tot)
        acc[1:2, :] += bM2 + delta * delta * (na * (BN_ / tot))


def _k1_body(xp, aggr, W1, b1, h1, stats, acc):
    i = pl.program_id(0)
    u = xp[...] + aggr[...]
    h = jnp.dot(u, W1[...], preferred_element_type=_f32) + b1[...]
    h1[...] = h
    _acc_stats(acc, h, i)

    @pl.when(i == NB - 1)
    def _():
        stats[...] = acc[...]


_k1 = pl.pallas_call(
    _k1_body,
    grid=(NB,),
    in_specs=[_blk(BN_, DP), _blk(BN_, DP), _rep(DP, DP), _rep(1, DP)],
    out_specs=[_blk(BN_, DP), _rep(8, DP)],
    out_shape=[jax.ShapeDtypeStruct((NNODE, DP), _f32),
               jax.ShapeDtypeStruct((8, DP), _f32)],
    scratch_shapes=[pltpu.VMEM((8, DP), _f32)],
    compiler_params=_ARB,
)


def _norm(stats, g, b):
    mu = stats[0:1, :]
    var = stats[1:2, :] * (1.0 / NNODE)
    scale = lax.rsqrt(var + 1e-5) * g
    return mu, scale, b


def _k2_body(h1, stats1, g1, be1, W2, b2, h2, stats, acc):
    i = pl.program_id(0)
    mu, scale, b = _norm(stats1[...], g1[...], be1[...])
    hn = jnp.maximum((h1[...] - mu) * scale + b, 0.0)
    h = jnp.dot(hn, W2[...], preferred_element_type=_f32) + b2[...]
    h2[...] = h
    _acc_stats(acc, h, i)

    @pl.when(i == NB - 1)
    def _():
        stats[...] = acc[...]


_k2 = pl.pallas_call(
    _k2_body,
    grid=(NB,),
    in_specs=[_blk(BN_, DP), _rep(8, DP), _rep(1, DP), _rep(1, DP),
              _rep(DP, DP), _rep(1, DP)],
    out_specs=[_blk(BN_, DP), _rep(8, DP)],
    out_shape=[jax.ShapeDtypeStruct((NNODE, DP), _f32),
               jax.ShapeDtypeStruct((8, DP), _f32)],
    scratch_shapes=[pltpu.VMEM((8, DP), _f32)],
    compiler_params=_ARB,
)


def _k3_body(h2, stats2, g2, be2, xp, oh, xo, pool, cnt, accp, accc):
    i = pl.program_id(0)
    mu, scale, b = _norm(stats2[...], g2[...], be2[...])
    xov = jnp.maximum((h2[...] - mu) * scale + b, 0.0) + xp[...]
    xo[...] = xov
    o = oh[...]

    @pl.when(i == 0)
    def _():
        accp[...] = jnp.zeros_like(accp)
        accc[...] = jnp.zeros_like(accc)

    accp[...] += lax.dot_general(o, xov, (((0,), (0,)), ((), ())),
                                 preferred_element_type=_f32,
                                 precision=lax.Precision.HIGHEST)
    accc[...] += lax.dot_general(o, jnp.ones((BN_, 8), _f32),
                                 (((0,), (0,)), ((), ())),
                                 preferred_element_type=_f32,
                                 precision=lax.Precision.HIGHEST)

    @pl.when(i == NB - 1)
    def _():
        pool[...] = accp[...]
        cnt[...] = accc[...]


_k3 = pl.pallas_call(
    _k3_body,
    grid=(NB,),
    in_specs=[_blk(BN_, DP), _rep(8, DP), _rep(1, DP), _rep(1, DP),
              _blk(BN_, DP), _blk(BN_, NG)],
    out_specs=[_blk(BN_, DP), _rep(NG, DP), _rep(NG, 8)],
    out_shape=[jax.ShapeDtypeStruct((NNODE, DP), _f32),
               jax.ShapeDtypeStruct((NG, DP), _f32),
               jax.ShapeDtypeStruct((NG, 8), _f32)],
    scratch_shapes=[pltpu.VMEM((NG, DP), _f32), pltpu.VMEM((NG, 8), _f32)],
    compiler_params=_ARB,
)


def _bn_relu_rows(h, g, b, n):
    mu = jnp.mean(h, axis=0, keepdims=True)
    var = jnp.mean((h - mu) ** 2, axis=0, keepdims=True)
    return jnp.maximum((h - mu) * lax.rsqrt(var + 1e-5) * g + b, 0.0)


def _k4_body(pool, cnt, vn, vW1, vb1, vg1, vbe1, vW2, vb2, vg2, vbe2, vnout):
    invc = 1.0 / jnp.maximum(cnt[...][:, 0:1], 1.0)
    vn2 = vn[...] + pool[...] * invc
    v = jnp.dot(vn2, vW1[...], preferred_element_type=_f32) + vb1[...]
    v = _bn_relu_rows(v, vg1[...], vbe1[...], NG)
    v = jnp.dot(v, vW2[...], preferred_element_type=_f32) + vb2[...]
    v = _bn_relu_rows(v, vg2[...], vbe2[...], NG)
    vnout[...] = v


_k4 = pl.pallas_call(
    _k4_body,
    out_shape=jax.ShapeDtypeStruct((NG, DP), _f32),
)


def _k5_body(xo, oh, vn3, xnext):
    xnext[...] = xo[...] + jnp.dot(oh[...], vn3[...],
                                   preferred_element_type=_f32,
                                   precision=lax.Precision.HIGHEST)


_k5 = pl.pallas_call(
    _k5_body,
    grid=(NB,),
    in_specs=[_blk(BN_, DP), _blk(BN_, NG), _rep(NG, DP)],
    out_specs=_blk(BN_, DP),
    out_shape=jax.ShapeDtypeStruct((NNODE, DP), _f32),
    compiler_params=_ARB,
)


def _k4f_body(pool, cnt, lW, lb, out):
    invc = 1.0 / jnp.maximum(cnt[...][:, 0:1], 1.0)
    out[...] = jnp.dot(pool[...] * invc, lW[...],
                       preferred_element_type=_f32) + lb[...]


_k4f = pl.pallas_call(
    _k4f_body,
    out_shape=jax.ShapeDtypeStruct((NG, 8), _f32),
)


def _pad_mat(w):
    return jnp.pad(w, ((0, DP - DIM), (0, DP - DIM)))


def _pad_vec(v):
    return jnp.pad(v, (0, DP - DIM))[None, :]


def kernel(x, edge_index, edge_attr, batch, params):
    p = params
    src = edge_index[0].astype(_i32)
    dst = edge_index[1].astype(_i32)
    combo = (edge_attr[:, 0] * 4 + edge_attr[:, 1] * 2
             + edge_attr[:, 2]).astype(_i32)
    perm = jnp.argsort(dst)
    srcs = src[perm]
    combos = combo[perm]
    dsts = dst[perm]
    dstl = (dsts % NPT).astype(_i32)
    ew = srcs * 4096 + combos * 512 + dstl
    bounds = (jnp.arange(1, NW + 1) * NPT).astype(_i32)
    starts = jnp.concatenate(
        [jnp.zeros((1,), _i32), jnp.searchsorted(dsts, bounds).astype(_i32)])
    starts = jnp.pad(starts, (0, 15))

    tab = (p['bond_t0'][:2, None, None, :] + p['bond_t1'][None, :2, None, :]
           + p['bond_t2'][None, None, :2, :]).reshape(8, DIM)
    tab = jnp.pad(tab, ((0, 0), (0, DP - DIM)))

    oh = (batch[:, None] == jnp.arange(NG, dtype=batch.dtype)[None, :]).astype(_f32)

    row0 = jnp.pad((p['const_x'] + p['vn_emb'])[0], (0, DP - DIM))
    xp = jnp.broadcast_to(row0, (NNODE, DP))

    vn = jnp.broadcast_to(jnp.pad(p['vn_emb'][0], (0, DP - DIM)), (NG, DP))

    for l in range(LAYERS):
        aggr = _edge_fn()(xp, ew, starts, tab)
        h1, st1 = _k1(xp, aggr[:NNODE], _pad_mat(p['conv_W1'][l]),
                      _pad_vec(p['conv_b1'][l]))
        h2, st2 = _k2(h1, st1, _pad_vec(p['conv_g1'][l]),
                      _pad_vec(p['conv_be1'][l]), _pad_mat(p['conv_W2'][l]),
                      _pad_vec(p['conv_b2'][l]))
        xo, pool, cnt = _k3(h2, st2, _pad_vec(p['bn_g'][l]),
                            _pad_vec(p['bn_b'][l]), xp, oh)
        if l < LAYERS - 1:
            vn = _k4(pool, cnt, vn,
                     _pad_mat(p['vn_W1'][l]), _pad_vec(p['vn_b1'][l]),
                     _pad_vec(p['vn_g1'][l]), _pad_vec(p['vn_be1'][l]),
                     _pad_mat(p['vn_W2'][l]), _pad_vec(p['vn_b2'][l]),
                     _pad_vec(p['vn_g2'][l]), _pad_vec(p['vn_be2'][l]))
            xp = _k5(xo, oh, vn)
        else:
            lw = jnp.pad(p['lin_W'], ((0, DP - DIM), (0, 7)))
            lb = jnp.pad(p['lin_b'], (0, 7))[None, :]
            out = _k4f(pool, cnt, lw, lb)
    return out[:, :1]
